# baseline (speedup 1.0000x reference)
import functools
import math

import jax
import jax.numpy as jnp
import numpy as np
from jax import lax
from jax.experimental import pallas as pl
from jax.experimental.pallas import tpu as pltpu
from jax.experimental.pallas import tpu_sc as plsc


def _pe_table(seq_len: int, d_model: int) -> np.ndarray:
    pos = np.arange(seq_len)[:, None].astype(np.float32)
    div = np.exp(
        np.arange(0, d_model, 2).astype(np.float32) * -(math.log(10000.0) / d_model)
    )
    pe = np.zeros((seq_len, d_model), dtype=np.float32)
    pe[:, 0::2] = np.sin(pos * div)
    pe[:, 1::2] = np.cos(pos * div)
    return pe


@functools.cache
def _build(batch: int, seq: int, vocab: int, d: int):
    n = batch * seq
    nc, ns, lanes = 2, 16, 16
    nw = nc * ns
    assert n % nw == 0
    per = n // nw
    chunk = 80
    assert per % chunk == 0 and d % lanes == 0
    nchunks = per // chunk
    scale = np.float32(np.sqrt(np.float32(d)))
    mesh = plsc.VectorSubcoreMesh(core_axis_name="c", subcore_axis_name="s")

    @functools.partial(
        pl.kernel,
        mesh=mesh,
        out_type=jax.ShapeDtypeStruct((n, d), jnp.float32),
        scratch_types=[
            pltpu.VMEM((nchunks, chunk), jnp.int32),
            pltpu.VMEM((seq, d), jnp.float32),
            pltpu.VMEM((chunk, d), jnp.float32),
            pltpu.SemaphoreType.DMA,
        ],
    )
    def emb(table, idx, pe, out, idx_v, pe_v, rows, sem):
        wid = lax.axis_index("s") * nc + lax.axis_index("c")
        pltpu.sync_copy(idx.at[wid], idx_v)
        pltpu.sync_copy(pe, pe_v)
        base = wid * per

        def chunk_body(j, carry):
            pltpu.async_copy(table.at[idx_v.at[j]], rows, sem).wait()

            def row_body(r, c2):
                p = lax.rem(j * chunk + r, seq)
                for k in range(d // lanes):
                    sl = pl.ds(k * lanes, lanes)
                    rows[r, sl] = rows[r, sl] * scale + pe_v[p, sl]
                return c2

            lax.fori_loop(0, chunk, row_body, 0)
            pltpu.sync_copy(rows, out.at[pl.ds(base + j * chunk, chunk)])
            return carry

        lax.fori_loop(0, nchunks, chunk_body, 0)

    return emb, nw, nchunks, chunk


def kernel(text_ids, emb_table):
    batch, seq = text_ids.shape
    vocab, d = emb_table.shape
    emb, nw, nchunks, chunk = _build(batch, seq, vocab, d)
    pe = jnp.asarray(_pe_table(seq, d))
    idx = text_ids.astype(jnp.int32).reshape(nw, nchunks, chunk)
    out = emb(emb_table, idx, pe)
    return out.reshape(batch, seq, d)


# speedup vs baseline: 1.2323x; 1.2323x over previous
import functools
import math

import jax
import jax.numpy as jnp
import numpy as np
from jax import lax
from jax.experimental import pallas as pl
from jax.experimental.pallas import tpu as pltpu
from jax.experimental.pallas import tpu_sc as plsc

_NBUF = 2


def _pe_table(seq_len: int, d_model: int) -> np.ndarray:
    pos = np.arange(seq_len)[:, None].astype(np.float32)
    div = np.exp(
        np.arange(0, d_model, 2).astype(np.float32) * -(math.log(10000.0) / d_model)
    )
    pe = np.zeros((seq_len, d_model), dtype=np.float32)
    pe[:, 0::2] = np.sin(pos * div)
    pe[:, 1::2] = np.cos(pos * div)
    return pe


@functools.cache
def _build(batch: int, seq: int, vocab: int, d: int):
    n = batch * seq
    nc, ns, lanes = 2, 16, 16
    nw = nc * ns
    assert n % nw == 0
    per = n // nw
    chunk = seq
    assert per % (chunk * _NBUF) == 0 and d % lanes == 0
    nchunks = per // chunk
    nsteps = nchunks // _NBUF
    scale = np.float32(np.sqrt(np.float32(d)))
    mesh = plsc.VectorSubcoreMesh(core_axis_name="c", subcore_axis_name="s")

    cd = chunk * d

    @functools.partial(
        pl.kernel,
        mesh=mesh,
        out_type=jax.ShapeDtypeStruct((n * d,), jnp.float32),
        scratch_types=[
            pltpu.VMEM((nchunks, chunk), jnp.int32),
            pltpu.VMEM((seq, d), jnp.float32),
            pltpu.VMEM((chunk, d), jnp.float32),
            pltpu.VMEM((chunk, d), jnp.float32),
            pltpu.VMEM((cd,), jnp.float32),
            pltpu.VMEM((cd,), jnp.float32),
            pltpu.SemaphoreType.DMA,
            pltpu.SemaphoreType.DMA,
            pltpu.SemaphoreType.DMA,
            pltpu.SemaphoreType.DMA,
        ],
    )
    def emb(table, idx, pe, out, idx_v, pe_v, g0, g1, o0, o1, gs0, gs1, os0, os1):
        wid = lax.axis_index("s") * nc + lax.axis_index("c")
        pltpu.sync_copy(idx.at[wid], idx_v)
        pltpu.sync_copy(pe, pe_v)
        base = wid * per * d
        gbufs, obufs = (g0, g1), (o0, o1)
        gsems, osems = (gs0, gs1), (os0, os1)

        for b in range(_NBUF):
            pltpu.async_copy(table.at[idx_v.at[b]], gbufs[b], gsems[b])

        def step(t, carry):
            for b in range(_NBUF):
                j = t * _NBUF + b
                gb, ob = gbufs[b], obufs[b]
                pltpu.make_async_copy(table.at[idx_v.at[0]], gb, gsems[b]).wait()

                @pl.when(t > 0)
                def _():
                    pltpu.make_async_copy(
                        ob, out.at[pl.ds(0, cd)], osems[b]
                    ).wait()

                for r in range(chunk):
                    for k in range(d // lanes):
                        sl = pl.ds(k * lanes, lanes)
                        ob[pl.ds((r * (d // lanes) + k) * lanes, lanes)] = (
                            gb[r, sl] * scale + pe_v[r, sl]
                        )

                @pl.when(t < nsteps - 1)
                def _():
                    pltpu.async_copy(table.at[idx_v.at[j + _NBUF]], gb, gsems[b])

                pltpu.async_copy(ob, out.at[pl.ds(base + j * cd, cd)], osems[b])
            return carry

        lax.fori_loop(0, nsteps, step, 0)
        for b in range(_NBUF):
            pltpu.make_async_copy(obufs[b], out.at[pl.ds(0, cd)], osems[b]).wait()

    return emb, nw, nchunks, chunk


def kernel(text_ids, emb_table):
    batch, seq = text_ids.shape
    vocab, d = emb_table.shape
    emb, nw, nchunks, chunk = _build(batch, seq, vocab, d)
    pe = jnp.asarray(_pe_table(seq, d))
    idx = text_ids.astype(jnp.int32).reshape(nw, nchunks, chunk)
    out = emb(emb_table, idx, pe)
    return out.reshape(batch, seq, d)


# speedup vs baseline: 1.6369x; 1.3283x over previous
import functools
import math

import jax
import jax.numpy as jnp
import numpy as np
from jax import lax
from jax.experimental import pallas as pl
from jax.experimental.pallas import tpu as pltpu
from jax.experimental.pallas import tpu_sc as plsc

_NBUF = 2


def _pe_table(seq_len: int, d_model: int) -> np.ndarray:
    pos = np.arange(seq_len)[:, None].astype(np.float32)
    div = np.exp(
        np.arange(0, d_model, 2).astype(np.float32) * -(math.log(10000.0) / d_model)
    )
    pe = np.zeros((seq_len, d_model), dtype=np.float32)
    pe[:, 0::2] = np.sin(pos * div)
    pe[:, 1::2] = np.cos(pos * div)
    return pe


@functools.cache
def _build(batch: int, seq: int, vocab: int, d: int):
    n = batch * seq
    nc, ns, lanes = 2, 16, 16
    nw = nc * ns
    assert n % nw == 0
    per = n // nw
    chunk = seq
    assert per % (chunk * _NBUF) == 0 and d % lanes == 0
    nchunks = per // chunk
    nsteps = nchunks // _NBUF
    scale = np.float32(np.sqrt(np.float32(d)))
    mesh = plsc.VectorSubcoreMesh(core_axis_name="c", subcore_axis_name="s")

    @functools.partial(
        pl.kernel,
        mesh=mesh,
        out_type=jax.ShapeDtypeStruct((batch, seq, d), jnp.float32),
        scratch_types=[
            pltpu.VMEM((nchunks, chunk), jnp.int32),
            pltpu.VMEM((seq, d), jnp.float32),
            pltpu.VMEM((chunk, d), jnp.float32),
            pltpu.VMEM((chunk, d), jnp.float32),
            pltpu.VMEM((chunk, d), jnp.float32),
            pltpu.VMEM((chunk, d), jnp.float32),
            pltpu.SemaphoreType.DMA,
            pltpu.SemaphoreType.DMA,
            pltpu.SemaphoreType.DMA,
            pltpu.SemaphoreType.DMA,
        ],
    )
    def emb(table, idx, pe, out, idx_v, pe_v, g0, g1, o0, o1, gs0, gs1, os0, os1):
        wid = lax.axis_index("s") * nc + lax.axis_index("c")
        pltpu.sync_copy(idx.at[wid], idx_v)
        pltpu.sync_copy(pe, pe_v)
        base_b = wid * nchunks
        gbufs, obufs = (g0, g1), (o0, o1)
        gsems, osems = (gs0, gs1), (os0, os1)

        for b in range(_NBUF):
            pltpu.async_copy(table.at[idx_v.at[b]], gbufs[b], gsems[b])

        def step(t, carry):
            for b in range(_NBUF):
                j = t * _NBUF + b
                gb, ob = gbufs[b], obufs[b]
                pltpu.make_async_copy(table.at[idx_v.at[0]], gb, gsems[b]).wait()

                @pl.when(t > 0)
                def _():
                    pltpu.make_async_copy(ob, out.at[0], osems[b]).wait()

                for r in range(chunk):
                    for k in range(d // lanes):
                        sl = pl.ds(k * lanes, lanes)
                        ob[r, sl] = gb[r, sl] * scale + pe_v[r, sl]

                @pl.when(t < nsteps - 1)
                def _():
                    pltpu.async_copy(table.at[idx_v.at[j + _NBUF]], gb, gsems[b])

                pltpu.async_copy(ob, out.at[base_b + j], osems[b])
            return carry

        lax.fori_loop(0, nsteps, step, 0)
        for b in range(_NBUF):
            pltpu.make_async_copy(obufs[b], out.at[0], osems[b]).wait()

    return emb, nw, nchunks, chunk


def kernel(text_ids, emb_table):
    batch, seq = text_ids.shape
    vocab, d = emb_table.shape
    emb, nw, nchunks, chunk = _build(batch, seq, vocab, d)
    pe = jnp.asarray(_pe_table(seq, d))
    idx = text_ids.astype(jnp.int32).reshape(nw, nchunks, chunk)
    return emb(emb_table, idx, pe)
